# Initial kernel scaffold; baseline (speedup 1.0000x reference)
import jax
import jax.numpy as jnp
from jax.experimental import pallas as pl


def kernel(x):
    raise NotImplementedError("write your pallas kernel here")


# speedup vs baseline: 22.0222x; 22.0222x over previous
import jax
import jax.numpy as jnp
from jax import lax
from jax.experimental import pallas as pl

_IN = (64, 96, 512)
_C0, _C1 = 128, 300
_W = _C1 - _C0
_R = _IN[0] * _IN[1]
_N = _R * _W
_K = int(0.1 * _N)
_CR, _CC = 4128, 256
_EPS_LOG = 0.001


def _select_norm_body(x_ref, o_ref):
    _I32_MIN = jnp.int32(-(2 ** 31))
    xs = x_ref[...]
    bits = lax.bitcast_convert_type(xs, jnp.int32)
    v = bits ^ (lax.shift_right_arithmetic(bits, 31) & jnp.int32(0x7FFFFFFF))

    def step(i, lo):
        mid = lo | lax.shift_left(jnp.int32(1), 31 - i)
        c = jnp.sum((v < (mid ^ _I32_MIN)).astype(jnp.int32))
        return jnp.where(c <= _K, mid, lo)

    lo = lax.fori_loop(0, 32, step, jnp.int32(0))
    vk = lo ^ _I32_MIN
    fb = vk ^ (lax.shift_right_arithmetic(vk, 31) & jnp.int32(0x7FFFFFFF))
    eps = lax.bitcast_convert_type(fb, jnp.float32)

    m = jnp.maximum(eps, jnp.float32(_EPS_LOG))
    xmax = jnp.max(xs)
    ylo = jnp.log10(m)
    yhi = jnp.log10(jnp.maximum(xmax, m))
    o_ref[...] = (jnp.log10(jnp.maximum(xs, m)) - ylo) / (yhi - ylo)


def kernel(x):
    xc = x.reshape(_R, _IN[2])[:, _C0:_C1].reshape(_CR, _CC)
    out = pl.pallas_call(
        _select_norm_body,
        out_shape=jax.ShapeDtypeStruct((_CR, _CC), jnp.float32),
    )(xc)
    return out.reshape(_IN[0], _IN[1], _W)
